# baseline (speedup 1.0000x reference)
import functools

import jax
import jax.numpy as jnp
from jax import lax
from jax.experimental import pallas as pl
from jax.experimental.pallas import tpu as pltpu
from jax.experimental.pallas import tpu_sc as plsc

_MAX_POS = 8192
_HIDDEN = 2048
_B, _S = 4, 4096
_N = _B * _S
_NC, _NS = 2, 16
_NW = _NC * _NS
_ROWS_PER_W = _N // _NW
_K = 16
_CHUNKS = _ROWS_PER_W // _K
_LANES = 16
_WORDS = _K * _HIDDEN // _LANES


def _pos_add_body(inp_hbm, idx_hbm, tab_hbm, out_hbm,
                  idx_v, bufx, buf, sem_in, sem_g):
    cid = lax.axis_index("c")
    sid = lax.axis_index("s")
    wid = sid * _NC + cid
    pltpu.sync_copy(idx_hbm.at[wid], idx_v)

    def chunk(c, carry):
        r0 = wid * _ROWS_PER_W + c * _K
        cp_in = pltpu.async_copy(inp_hbm.at[pl.ds(r0, _K)], bufx, sem_in)
        cp_g = pltpu.async_copy(tab_hbm.at[idx_v.at[c]], buf, sem_g)
        cp_in.wait()
        cp_g.wait()

        def add_row(r, carry2):
            def add_word(j, carry3):
                sl = pl.ds(j * _LANES, _LANES)
                plsc.addupdate(buf.at[r, sl], bufx[r, sl])
                return carry3

            lax.fori_loop(0, _HIDDEN // _LANES, add_word, 0, unroll=8)
            return carry2

        lax.fori_loop(0, _K, add_row, 0)
        pltpu.sync_copy(buf, out_hbm.at[pl.ds(r0, _K)])
        return carry

    lax.fori_loop(0, _CHUNKS, chunk, 0)


@jax.jit
def _pos_add(inp, idx, tab):
    mesh = plsc.VectorSubcoreMesh(core_axis_name="c", subcore_axis_name="s")
    f = pl.kernel(
        _pos_add_body,
        out_type=jax.ShapeDtypeStruct((_N, _HIDDEN), jnp.float32),
        mesh=mesh,
        scratch_types=[
            pltpu.VMEM((_CHUNKS, _K), jnp.int32),
            pltpu.VMEM((_K, _HIDDEN), jnp.float32),
            pltpu.VMEM((_K, _HIDDEN), jnp.float32),
            pltpu.SemaphoreType.DMA,
            pltpu.SemaphoreType.DMA,
        ],
    )
    return f(inp, idx, tab)


def kernel(input_ids, position_ids, pos_table):
    inp = input_ids.reshape(_N, _HIDDEN)
    idx = position_ids.astype(jnp.int32).reshape(_NW, _CHUNKS, _K)
    out = _pos_add(inp, idx, pos_table)
    return out.reshape(_B, _S, _HIDDEN)


# speedup vs baseline: 2.7513x; 2.7513x over previous
import functools

import jax
import jax.numpy as jnp
from jax import lax
from jax.experimental import pallas as pl
from jax.experimental.pallas import tpu as pltpu
from jax.experimental.pallas import tpu_sc as plsc

_MAX_POS = 8192
_HIDDEN = 2048
_B, _S = 4, 4096
_N = _B * _S
_NC, _NS = 2, 16
_NW = _NC * _NS
_ROWS_PER_W = _N // _NW
_K = 8
_CHUNKS = _ROWS_PER_W // _K
_LANES = 16
_NT = 4
_NX = 2


def _pos_add_body(inp_hbm, idx_hbm, tab_hbm, out_hbm, idx_v,
                  x0, x1, t0, t1, t2, t3,
                  si0, si1, sg0, sg1, sg2, sg3, so0, so1, so2, so3):
    X = [x0, x1]
    T = [t0, t1, t2, t3]
    SI = [si0, si1]
    SG = [sg0, sg1, sg2, sg3]
    SO = [so0, so1, so2, so3]

    wid = lax.axis_index("s") * _NC + lax.axis_index("c")
    base = wid * _ROWS_PER_W
    pltpu.sync_copy(idx_hbm.at[wid], idx_v)

    def start_in(c, bx):
        pltpu.async_copy(inp_hbm.at[pl.ds(base + c * _K, _K)], X[bx], SI[bx])

    def start_gather(c, bt):
        pltpu.async_copy(tab_hbm.at[idx_v.at[c]], T[bt], SG[bt])

    start_in(0, 0)
    start_gather(0, 0)
    start_in(1, 1)
    start_gather(1, 1)

    def group(g, carry):
        for b in range(_NT):
            c = g * _NT + b
            bx = b % _NX
            pltpu.make_async_copy(
                inp_hbm.at[pl.ds(base, _K)], X[bx], SI[bx]).wait()
            pltpu.make_async_copy(
                tab_hbm.at[idx_v.at[0]], T[b], SG[b]).wait()

            def add_row(r, car):
                def add_word(j, car2):
                    sl = pl.ds(j * _LANES, _LANES)
                    plsc.addupdate(T[b].at[r, sl], X[bx][r, sl])
                    return car2

                lax.fori_loop(0, _HIDDEN // _LANES, add_word, 0, unroll=8)
                return car

            lax.fori_loop(0, _K, add_row, 0)
            pltpu.async_copy(T[b], out_hbm.at[pl.ds(base + c * _K, _K)], SO[b])

            bt2 = (b + _NX) % _NT

            @pl.when(c < _CHUNKS - _NX)
            def _():
                start_in(c + _NX, bx)

                @pl.when(c >= _NX)
                def _():
                    pltpu.make_async_copy(
                        T[bt2], out_hbm.at[pl.ds(base, _K)], SO[bt2]).wait()

                start_gather(c + _NX, bt2)
        return carry

    lax.fori_loop(0, _CHUNKS // _NT, group, 0)
    for b in range(_NT):
        pltpu.make_async_copy(
            T[b], out_hbm.at[pl.ds(base, _K)], SO[b]).wait()


@jax.jit
def _pos_add(inp, idx, tab):
    mesh = plsc.VectorSubcoreMesh(core_axis_name="c", subcore_axis_name="s")
    f = pl.kernel(
        _pos_add_body,
        out_type=jax.ShapeDtypeStruct((_N, _HIDDEN), jnp.float32),
        mesh=mesh,
        scratch_types=(
            [pltpu.VMEM((_CHUNKS, _K), jnp.int32)]
            + [pltpu.VMEM((_K, _HIDDEN), jnp.float32)] * (_NX + _NT)
            + [pltpu.SemaphoreType.DMA] * (_NX + 2 * _NT)
        ),
    )
    return f(inp, idx, tab)


def kernel(input_ids, position_ids, pos_table):
    inp = input_ids.reshape(_N, _HIDDEN)
    idx = position_ids.astype(jnp.int32).reshape(_NW, _CHUNKS, _K)
    out = _pos_add(inp, idx, pos_table)
    return out.reshape(_B, _S, _HIDDEN)


# speedup vs baseline: 2.7578x; 1.0024x over previous
import functools

import jax
import jax.numpy as jnp
from jax import lax
from jax.experimental import pallas as pl
from jax.experimental.pallas import tpu as pltpu
from jax.experimental.pallas import tpu_sc as plsc

_MAX_POS = 8192
_HIDDEN = 2048
_B, _S = 4, 4096
_N = _B * _S
_NC, _NS = 2, 16
_NW = _NC * _NS
_ROWS_PER_W = _N // _NW
_K = 8
_CHUNKS = _ROWS_PER_W // _K
_LANES = 16
_NT = 4
_NX = 2


def _pos_add_body(inp_hbm, idx_hbm, tab_hbm, out_hbm, idx_v,
                  x0, x1, t0, t1, t2, t3,
                  si0, si1, sg0, sg1, sg2, sg3, so0, so1, so2, so3):
    X = [x0, x1]
    T = [t0, t1, t2, t3]
    SI = [si0, si1]
    SG = [sg0, sg1, sg2, sg3]
    SO = [so0, so1, so2, so3]

    wid = lax.axis_index("s") * _NC + lax.axis_index("c")
    base = wid * _ROWS_PER_W
    pltpu.sync_copy(idx_hbm.at[wid], idx_v)

    def start_in(c, bx):
        pltpu.async_copy(inp_hbm.at[pl.ds(base + c * _K, _K)], X[bx], SI[bx])

    def start_gather(c, bt):
        pltpu.async_copy(tab_hbm.at[idx_v.at[c]], T[bt], SG[bt])

    start_in(0, 0)
    start_gather(0, 0)
    start_in(1, 1)
    start_gather(1, 1)

    def group(g, carry):
        for b in range(_NT):
            c = g * _NT + b
            bx = b % _NX
            pltpu.make_async_copy(
                inp_hbm.at[pl.ds(base, _K)], X[bx], SI[bx]).wait()
            pltpu.make_async_copy(
                tab_hbm.at[idx_v.at[0]], T[b], SG[b]).wait()

            bt2 = (b + _NX) % _NT

            @pl.when(c < _CHUNKS - _NX)
            def _():
                @pl.when(c >= _NX)
                def _():
                    pltpu.make_async_copy(
                        T[bt2], out_hbm.at[pl.ds(base, _K)], SO[bt2]).wait()

                start_gather(c + _NX, bt2)

            def add_row(r, car):
                def add_word(j, car2):
                    sl = pl.ds(j * _LANES, _LANES)
                    plsc.addupdate(T[b].at[r, sl], X[bx][r, sl])
                    return car2

                lax.fori_loop(0, _HIDDEN // _LANES, add_word, 0, unroll=16)
                return car

            lax.fori_loop(0, _K, add_row, 0)
            pltpu.async_copy(T[b], out_hbm.at[pl.ds(base + c * _K, _K)], SO[b])

            @pl.when(c < _CHUNKS - _NX)
            def _():
                start_in(c + _NX, bx)
        return carry

    lax.fori_loop(0, _CHUNKS // _NT, group, 0)
    for b in range(_NT):
        pltpu.make_async_copy(
            T[b], out_hbm.at[pl.ds(base, _K)], SO[b]).wait()


@jax.jit
def _pos_add(inp, idx, tab):
    mesh = plsc.VectorSubcoreMesh(core_axis_name="c", subcore_axis_name="s")
    f = pl.kernel(
        _pos_add_body,
        out_type=jax.ShapeDtypeStruct((_N, _HIDDEN), jnp.float32),
        mesh=mesh,
        scratch_types=(
            [pltpu.VMEM((_CHUNKS, _K), jnp.int32)]
            + [pltpu.VMEM((_K, _HIDDEN), jnp.float32)] * (_NX + _NT)
            + [pltpu.SemaphoreType.DMA] * (_NX + 2 * _NT)
        ),
    )
    return f(inp, idx, tab)


def kernel(input_ids, position_ids, pos_table):
    inp = input_ids.reshape(_N, _HIDDEN)
    idx = position_ids.astype(jnp.int32).reshape(_NW, _CHUNKS, _K)
    out = _pos_add(inp, idx, pos_table)
    return out.reshape(_B, _S, _HIDDEN)


# speedup vs baseline: 2.7768x; 1.0069x over previous
import functools

import jax
import jax.numpy as jnp
from jax import lax
from jax.experimental import pallas as pl
from jax.experimental.pallas import tpu as pltpu
from jax.experimental.pallas import tpu_sc as plsc

_MAX_POS = 8192
_HIDDEN = 2048
_B, _S = 4, 4096
_N = _B * _S
_NC, _NS = 2, 16
_NW = _NC * _NS
_ROWS_PER_W = _N // _NW
_K = 4
_CHUNKS = _ROWS_PER_W // _K
_LANES = 16
_NT = 8
_NX = 4


def _pos_add_body(inp_hbm, idx_hbm, tab_hbm, out_hbm, idx_v,
                  x0, x1, x2, x3, t0, t1, t2, t3, t4, t5, t6, t7,
                  si0, si1, si2, si3,
                  sg0, sg1, sg2, sg3, sg4, sg5, sg6, sg7,
                  so0, so1, so2, so3, so4, so5, so6, so7):
    X = [x0, x1, x2, x3]
    T = [t0, t1, t2, t3, t4, t5, t6, t7]
    SI = [si0, si1, si2, si3]
    SG = [sg0, sg1, sg2, sg3, sg4, sg5, sg6, sg7]
    SO = [so0, so1, so2, so3, so4, so5, so6, so7]

    wid = lax.axis_index("s") * _NC + lax.axis_index("c")
    base = wid * _ROWS_PER_W
    pltpu.sync_copy(idx_hbm.at[wid], idx_v)

    def start_in(c, bx):
        pltpu.async_copy(inp_hbm.at[pl.ds(base + c * _K, _K)], X[bx], SI[bx])

    def start_gather(c, bt):
        pltpu.async_copy(tab_hbm.at[idx_v.at[c]], T[bt], SG[bt])

    for p in range(_NX):
        start_in(p, p)
        start_gather(p, p)

    def group(g, carry):
        for b in range(_NT):
            c = g * _NT + b
            bx = b % _NX
            pltpu.make_async_copy(
                inp_hbm.at[pl.ds(base, _K)], X[bx], SI[bx]).wait()
            pltpu.make_async_copy(
                tab_hbm.at[idx_v.at[0]], T[b], SG[b]).wait()

            bt2 = (b + _NX) % _NT

            @pl.when(c < _CHUNKS - _NX)
            def _():
                @pl.when(c >= _NX)
                def _():
                    pltpu.make_async_copy(
                        T[bt2], out_hbm.at[pl.ds(base, _K)], SO[bt2]).wait()

                start_gather(c + _NX, bt2)

            def add_row(r, car):
                def add_word(j, car2):
                    sl = pl.ds(j * _LANES, _LANES)
                    plsc.addupdate(T[b].at[r, sl], X[bx][r, sl])
                    return car2

                lax.fori_loop(0, _HIDDEN // _LANES, add_word, 0, unroll=16)
                return car

            lax.fori_loop(0, _K, add_row, 0)
            pltpu.async_copy(T[b], out_hbm.at[pl.ds(base + c * _K, _K)], SO[b])

            @pl.when(c < _CHUNKS - _NX)
            def _():
                start_in(c + _NX, bx)
        return carry

    lax.fori_loop(0, _CHUNKS // _NT, group, 0)
    for b in range(_NT):
        pltpu.make_async_copy(
            T[b], out_hbm.at[pl.ds(base, _K)], SO[b]).wait()


@jax.jit
def _pos_add(inp, idx, tab):
    mesh = plsc.VectorSubcoreMesh(core_axis_name="c", subcore_axis_name="s")
    f = pl.kernel(
        _pos_add_body,
        out_type=jax.ShapeDtypeStruct((_N, _HIDDEN), jnp.float32),
        mesh=mesh,
        scratch_types=(
            [pltpu.VMEM((_CHUNKS, _K), jnp.int32)]
            + [pltpu.VMEM((_K, _HIDDEN), jnp.float32)] * (_NX + _NT)
            + [pltpu.SemaphoreType.DMA] * (_NX + 2 * _NT)
        ),
    )
    return f(inp, idx, tab)


def kernel(input_ids, position_ids, pos_table):
    inp = input_ids.reshape(_N, _HIDDEN)
    idx = position_ids.astype(jnp.int32).reshape(_NW, _CHUNKS, _K)
    out = _pos_add(inp, idx, pos_table)
    return out.reshape(_B, _S, _HIDDEN)
